# Initial kernel scaffold; baseline (speedup 1.0000x reference)
import functools

import jax
import jax.numpy as jnp
from jax import lax
from jax.experimental import pallas as pl
from jax.experimental.pallas import tpu as pltpu
from jax.experimental.pallas import tpu_sc as plsc

N = 10000
E = 320000
D = 128

NC = 2
NS = 16
NW = NC * NS
EPW = E // NW
CHUNK = 80
ROWS_PER_TILE = N // NS


def _sc_agg_body(x_hbm, src_hbm, dst_hbm, parts_hbm, agg_sh, sidx, didx,
                 rows, sem):
    c = lax.axis_index("c")
    s = lax.axis_index("s")
    rbase = s * ROWS_PER_TILE
    pltpu.sync_copy(x_hbm.at[pl.ds(rbase, ROWS_PER_TILE)],
                    agg_sh.at[pl.ds(rbase, ROWS_PER_TILE)])
    plsc.subcore_barrier()

    tile_base = (c * NS + s) * EPW

    def body(i, carry):
        ebase = pl.multiple_of(tile_base + i * CHUNK, 8)
        pltpu.sync_copy(src_hbm.at[pl.ds(ebase, CHUNK)], sidx)
        pltpu.sync_copy(dst_hbm.at[pl.ds(ebase, CHUNK)], didx)
        pltpu.async_copy(x_hbm.at[sidx], rows, sem).wait()
        pltpu.sync_copy(rows, agg_sh.at[didx], add=True)
        return carry

    lax.fori_loop(0, EPW // CHUNK, body, 0)
    plsc.subcore_barrier()
    pltpu.sync_copy(agg_sh.at[pl.ds(rbase, ROWS_PER_TILE)],
                    parts_hbm.at[c, pl.ds(rbase, ROWS_PER_TILE)])


_sc_agg = pl.kernel(
    _sc_agg_body,
    out_type=jax.ShapeDtypeStruct((NC, N, D), jnp.float32),
    mesh=plsc.VectorSubcoreMesh(core_axis_name="c", subcore_axis_name="s"),
    scratch_types=[
        pltpu.VMEM_SHARED((N, D), jnp.float32),
        pltpu.VMEM((CHUNK,), jnp.int32),
        pltpu.VMEM((CHUNK,), jnp.int32),
        pltpu.VMEM((CHUNK, D), jnp.float32),
        pltpu.SemaphoreType.DMA,
    ],
)


BLK = 1000
NB = N // BLK


def _tc_mlp_bn_body(a0, a1, x, W1, W2, b1, b2, gamma, beta, y, h_s, stat_s):
    p = pl.program_id(0)
    j = pl.program_id(1)

    @pl.when(p == 0)
    def _phase0():
        hin = a0[...] + a1[...] - x[...]
        m = jnp.dot(hin, W1[...], preferred_element_type=jnp.float32)
        m = jnp.maximum(m + b1[...], 0.0)
        h = jnp.dot(m, W2[...], preferred_element_type=jnp.float32)
        h = jnp.maximum(h + b2[...], 0.0)
        h_s[pl.ds(j * BLK, BLK), :] = h
        bs = jnp.sum(h, axis=0, keepdims=True)
        bq = jnp.sum(h * h, axis=0, keepdims=True)

        @pl.when(j == 0)
        def _init():
            stat_s[0:1, :] = bs
            stat_s[1:2, :] = bq

        @pl.when(j > 0)
        def _acc():
            stat_s[0:1, :] += bs
            stat_s[1:2, :] += bq

        y[...] = h

    @pl.when(p == 1)
    def _phase1():
        mean = stat_s[0:1, :] * (1.0 / N)
        var = stat_s[1:2, :] * (1.0 / N) - mean * mean
        rstd = lax.rsqrt(var + 1e-5)
        h = h_s[pl.ds(j * BLK, BLK), :]
        y[...] = (h - mean) * (rstd * gamma[...]) + beta[...]


def _tc_mlp_bn(a0, a1, x, W1, W2, b1, b2, gamma, beta):
    row_spec = pl.BlockSpec((BLK, D), lambda p, j: (j, 0))
    mat_spec = pl.BlockSpec((D, D), lambda p, j: (0, 0))
    vec_spec = pl.BlockSpec((1, D), lambda p, j: (0, 0))
    return pl.pallas_call(
        _tc_mlp_bn_body,
        grid=(2, NB),
        in_specs=[row_spec, row_spec, row_spec, mat_spec, mat_spec,
                  vec_spec, vec_spec, vec_spec, vec_spec],
        out_specs=row_spec,
        out_shape=jax.ShapeDtypeStruct((N, D), jnp.float32),
        scratch_shapes=[
            pltpu.VMEM((N, D), jnp.float32),
            pltpu.VMEM((2, D), jnp.float32),
        ],
        compiler_params=pltpu.CompilerParams(
            dimension_semantics=("arbitrary", "arbitrary")),
    )(a0, a1, x, W1, W2, b1, b2, gamma, beta)


def kernel(x, edge_index, batch, W1, b1, W2, b2, gamma, beta):
    src = edge_index[0].astype(jnp.int32)
    dst = edge_index[1].astype(jnp.int32)
    parts = _sc_agg(x, src, dst)
    return _tc_mlp_bn(parts[0], parts[1], x, W1, W2,
                      b1.reshape(1, D), b2.reshape(1, D),
                      gamma.reshape(1, D), beta.reshape(1, D))


# speedup vs baseline: 4.7248x; 4.7248x over previous
import functools

import jax
import jax.numpy as jnp
from jax import lax
from jax.experimental import pallas as pl
from jax.experimental.pallas import tpu as pltpu
from jax.experimental.pallas import tpu_sc as plsc

N = 10000
E = 320000
D = 128

NC = 2
NS = 16
NW = NC * NS
EPW = E // NW
CHUNK = 80
NPAD = 10240
ROWS_PER_TILE = NPAD // NS


def _sc_agg_body(x_hbm, src_hbm, dst_hbm, parts_hbm, agg_sh, sidx, didx,
                 rows, sem):
    c = lax.axis_index("c")
    s = lax.axis_index("s")
    rbase = s * ROWS_PER_TILE
    pltpu.sync_copy(x_hbm.at[pl.ds(rbase, ROWS_PER_TILE)],
                    agg_sh.at[pl.ds(rbase, ROWS_PER_TILE)])
    plsc.subcore_barrier()

    tile_base = (c * NS + s) * EPW

    def body(i, carry):
        ebase = pl.multiple_of(tile_base + i * CHUNK, 8)
        pltpu.sync_copy(src_hbm.at[pl.ds(ebase, CHUNK)], sidx)
        pltpu.sync_copy(dst_hbm.at[pl.ds(ebase, CHUNK)], didx)
        pltpu.async_copy(x_hbm.at[sidx], rows, sem).wait()
        pltpu.sync_copy(rows, agg_sh.at[didx], add=True)
        return carry

    lax.fori_loop(0, EPW // CHUNK, body, 0)
    plsc.subcore_barrier()
    pltpu.sync_copy(agg_sh.at[pl.ds(rbase, ROWS_PER_TILE)],
                    parts_hbm.at[c, pl.ds(rbase, ROWS_PER_TILE)])


_sc_agg = pl.kernel(
    _sc_agg_body,
    out_type=jax.ShapeDtypeStruct((NC, NPAD, D), jnp.float32),
    mesh=plsc.VectorSubcoreMesh(core_axis_name="c", subcore_axis_name="s"),
    scratch_types=[
        pltpu.VMEM_SHARED((NPAD, D), jnp.float32),
        pltpu.VMEM((CHUNK,), jnp.int32),
        pltpu.VMEM((CHUNK,), jnp.int32),
        pltpu.VMEM((CHUNK, D), jnp.float32),
        pltpu.SemaphoreType.DMA,
    ],
)


BLK = 1000
NB = N // BLK


def _tc_mlp_bn_body(a0, a1, x, W1, W2, b1, b2, gamma, beta, y, h_s, stat_s):
    p = pl.program_id(0)
    j = pl.program_id(1)

    @pl.when(p == 0)
    def _phase0():
        hin = a0[...] + a1[...] - x[...]
        m = jnp.dot(hin, W1[...], preferred_element_type=jnp.float32)
        m = jnp.maximum(m + b1[...], 0.0)
        h = jnp.dot(m, W2[...], preferred_element_type=jnp.float32)
        h = jnp.maximum(h + b2[...], 0.0)
        h_s[pl.ds(j * BLK, BLK), :] = h
        bs = jnp.sum(h, axis=0, keepdims=True)
        bq = jnp.sum(h * h, axis=0, keepdims=True)

        @pl.when(j == 0)
        def _init():
            stat_s[0:1, :] = bs
            stat_s[1:2, :] = bq

        @pl.when(j > 0)
        def _acc():
            stat_s[0:1, :] += bs
            stat_s[1:2, :] += bq

        y[...] = h

    @pl.when(p == 1)
    def _phase1():
        mean = stat_s[0:1, :] * (1.0 / N)
        var = stat_s[1:2, :] * (1.0 / N) - mean * mean
        rstd = lax.rsqrt(var + 1e-5)
        h = h_s[pl.ds(j * BLK, BLK), :]
        y[...] = (h - mean) * (rstd * gamma[...]) + beta[...]


def _tc_mlp_bn(a0, a1, x, W1, W2, b1, b2, gamma, beta):
    row_spec = pl.BlockSpec((BLK, D), lambda p, j: (j, 0))
    mat_spec = pl.BlockSpec((D, D), lambda p, j: (0, 0))
    vec_spec = pl.BlockSpec((1, D), lambda p, j: (0, 0))
    return pl.pallas_call(
        _tc_mlp_bn_body,
        grid=(2, NB),
        in_specs=[row_spec, row_spec, row_spec, mat_spec, mat_spec,
                  vec_spec, vec_spec, vec_spec, vec_spec],
        out_specs=row_spec,
        out_shape=jax.ShapeDtypeStruct((N, D), jnp.float32),
        scratch_shapes=[
            pltpu.VMEM((N, D), jnp.float32),
            pltpu.VMEM((2, D), jnp.float32),
        ],
        compiler_params=pltpu.CompilerParams(
            dimension_semantics=("arbitrary", "arbitrary")),
    )(a0, a1, x, W1, W2, b1, b2, gamma, beta)


def kernel(x, edge_index, batch, W1, b1, W2, b2, gamma, beta):
    src = edge_index[0].astype(jnp.int32)
    dst = edge_index[1].astype(jnp.int32)
    x_pad = jnp.pad(x, ((0, NPAD - N), (0, 0)))
    parts = _sc_agg(x_pad, src, dst)
    return _tc_mlp_bn(parts[0], parts[1], x, W1, W2,
                      b1.reshape(1, D), b2.reshape(1, D),
                      gamma.reshape(1, D), beta.reshape(1, D))


# speedup vs baseline: 7.4795x; 1.5830x over previous
import jax
import jax.numpy as jnp
from jax import lax
from jax.experimental import pallas as pl
from jax.experimental.pallas import tpu as pltpu
from jax.experimental.pallas import tpu_sc as plsc

N = 10000
E = 320000
D = 128

NC = 2
NS = 16
NW = NC * NS
EPW = E // NW
CHUNK = 80
ITERS = EPW // CHUNK
NPAD = 10240
ROWS_PER_TILE = NPAD // NS
LAST_TILE_ROWS = N - (NS - 1) * ROWS_PER_TILE


def _sc_agg_body(x_hbm, src_hbm, dst_hbm, parts_hbm, agg_sh, sidx0, didx0,
                 sidx1, didx1, rows0, rows1, sem0, sem1):
    c = lax.axis_index("c")
    s = lax.axis_index("s")
    w = c * NS + s
    rbase = s * ROWS_PER_TILE

    @pl.when(s < NS - 1)
    def _init_full():
        pltpu.sync_copy(x_hbm.at[pl.ds(rbase, ROWS_PER_TILE)],
                        agg_sh.at[pl.ds(rbase, ROWS_PER_TILE)])

    @pl.when(s == NS - 1)
    def _init_last():
        pltpu.sync_copy(x_hbm.at[pl.ds((NS - 1) * ROWS_PER_TILE,
                                       LAST_TILE_ROWS)],
                        agg_sh.at[pl.ds((NS - 1) * ROWS_PER_TILE,
                                        LAST_TILE_ROWS)])

    plsc.subcore_barrier()

    tile_base = w * EPW

    def load_idx(i, sbuf, dbuf):
        ebase = pl.multiple_of(tile_base + i * CHUNK, 8)
        pltpu.sync_copy(src_hbm.at[pl.ds(ebase, CHUNK)], sbuf)
        pltpu.sync_copy(dst_hbm.at[pl.ds(ebase, CHUNK)], dbuf)

    def gather(sbuf, buf, sem):
        return pltpu.make_async_copy(x_hbm.at[sbuf], buf, sem)

    def scatter(dbuf, buf):
        pltpu.sync_copy(buf, agg_sh.at[dbuf], add=True)

    load_idx(0, sidx0, didx0)
    gather(sidx0, rows0, sem0).start()

    def pair(p, carry):
        i0 = 2 * p
        load_idx(i0 + 1, sidx1, didx1)
        gather(sidx1, rows1, sem1).start()
        gather(sidx0, rows0, sem0).wait()
        scatter(didx0, rows0)
        load_idx(i0 + 2, sidx0, didx0)
        gather(sidx0, rows0, sem0).start()
        gather(sidx1, rows1, sem1).wait()
        scatter(didx1, rows1)
        return carry

    lax.fori_loop(0, (ITERS - 1) // 2, pair, 0)
    gather(sidx0, rows0, sem0).wait()
    scatter(didx0, rows0)

    plsc.subcore_barrier()

    @pl.when(s < NS - 1)
    def _exp_full():
        pltpu.sync_copy(agg_sh.at[pl.ds(rbase, ROWS_PER_TILE)],
                        parts_hbm.at[c, pl.ds(rbase, ROWS_PER_TILE)])

    @pl.when(s == NS - 1)
    def _exp_last():
        pltpu.sync_copy(agg_sh.at[pl.ds((NS - 1) * ROWS_PER_TILE,
                                        LAST_TILE_ROWS)],
                        parts_hbm.at[c, pl.ds((NS - 1) * ROWS_PER_TILE,
                                              LAST_TILE_ROWS)])


_sc_agg = pl.kernel(
    _sc_agg_body,
    out_type=jax.ShapeDtypeStruct((NC, NPAD, D), jnp.float32),
    mesh=plsc.VectorSubcoreMesh(core_axis_name="c", subcore_axis_name="s"),
    scratch_types=[
        pltpu.VMEM_SHARED((NPAD, D), jnp.float32),
        pltpu.VMEM((CHUNK,), jnp.int32),
        pltpu.VMEM((CHUNK,), jnp.int32),
        pltpu.VMEM((CHUNK,), jnp.int32),
        pltpu.VMEM((CHUNK,), jnp.int32),
        pltpu.VMEM((CHUNK, D), jnp.float32),
        pltpu.VMEM((CHUNK, D), jnp.float32),
        pltpu.SemaphoreType.DMA,
        pltpu.SemaphoreType.DMA,
    ],
)


BLK = 1000
NB = N // BLK


def _tc_mlp_bn_body(parts, x, W1, W2, b1, b2, gamma, beta, y, h_s, stat_s):
    p = pl.program_id(0)
    j = pl.program_id(1)

    @pl.when(p == 0)
    def _phase0():
        hin = parts[0] + parts[1] - x[...]
        m = jnp.dot(hin, W1[...], preferred_element_type=jnp.float32)
        m = jnp.maximum(m + b1[...], 0.0)
        h = jnp.dot(m, W2[...], preferred_element_type=jnp.float32)
        h = jnp.maximum(h + b2[...], 0.0)
        h_s[pl.ds(j * BLK, BLK), :] = h
        bs = jnp.sum(h, axis=0, keepdims=True)
        bq = jnp.sum(h * h, axis=0, keepdims=True)

        @pl.when(j == 0)
        def _init():
            stat_s[0:1, :] = bs
            stat_s[1:2, :] = bq

        @pl.when(j > 0)
        def _acc():
            stat_s[0:1, :] += bs
            stat_s[1:2, :] += bq

        y[...] = h

    @pl.when(p == 1)
    def _phase1():
        mean = stat_s[0:1, :] * (1.0 / N)
        var = stat_s[1:2, :] * (1.0 / N) - mean * mean
        rstd = lax.rsqrt(var + 1e-5)
        h = h_s[pl.ds(j * BLK, BLK), :]
        y[...] = (h - mean) * (rstd * gamma[...]) + beta[...]


def _tc_mlp_bn(parts, x, W1, W2, b1, b2, gamma, beta):
    row_spec = pl.BlockSpec((BLK, D), lambda p, j: (j, 0))
    mat_spec = pl.BlockSpec((D, D), lambda p, j: (0, 0))
    vec_spec = pl.BlockSpec((1, D), lambda p, j: (0, 0))
    parts_spec = pl.BlockSpec((NC, BLK, D), lambda p, j: (0, j, 0))
    return pl.pallas_call(
        _tc_mlp_bn_body,
        grid=(2, NB),
        in_specs=[parts_spec, row_spec, mat_spec, mat_spec,
                  vec_spec, vec_spec, vec_spec, vec_spec],
        out_specs=row_spec,
        out_shape=jax.ShapeDtypeStruct((N, D), jnp.float32),
        scratch_shapes=[
            pltpu.VMEM((N, D), jnp.float32),
            pltpu.VMEM((2, D), jnp.float32),
        ],
        compiler_params=pltpu.CompilerParams(
            dimension_semantics=("arbitrary", "arbitrary")),
    )(parts, x, W1, W2, b1, b2, gamma, beta)


def kernel(x, edge_index, batch, W1, b1, W2, b2, gamma, beta):
    src = edge_index[0].astype(jnp.int32)
    dst = edge_index[1].astype(jnp.int32)
    parts = _sc_agg(x, src, dst)
    return _tc_mlp_bn(parts, x, W1, W2,
                      b1.reshape(1, D), b2.reshape(1, D),
                      gamma.reshape(1, D), beta.reshape(1, D))


# speedup vs baseline: 10.1762x; 1.3605x over previous
import jax
import jax.numpy as jnp
from jax import lax
from jax.experimental import pallas as pl
from jax.experimental.pallas import tpu as pltpu
from jax.experimental.pallas import tpu_sc as plsc

N = 10000
E = 320000
D = 128

NC = 2
NS = 16
NW = NC * NS
EPW = E // NW
CHUNK = 80
ITERS = EPW // CHUNK
NPAD = 10240
ROWS_PER_TILE = NPAD // NS
LAST_TILE_ROWS = N - (NS - 1) * ROWS_PER_TILE


def _sc_agg_body(x_hbm, src_hbm, dst_hbm, parts_hbm, agg_sh, sidx_all,
                 didx0, didx1, rows0, rows1, semg0, semg1, semi0, semi1):
    c = lax.axis_index("c")
    s = lax.axis_index("s")
    w = c * NS + s
    rbase = s * ROWS_PER_TILE

    @pl.when(s < NS - 1)
    def _init_full():
        pltpu.sync_copy(x_hbm.at[pl.ds(rbase, ROWS_PER_TILE)],
                        agg_sh.at[pl.ds(rbase, ROWS_PER_TILE)])

    @pl.when(s == NS - 1)
    def _init_last():
        pltpu.sync_copy(x_hbm.at[pl.ds((NS - 1) * ROWS_PER_TILE,
                                       LAST_TILE_ROWS)],
                        agg_sh.at[pl.ds((NS - 1) * ROWS_PER_TILE,
                                        LAST_TILE_ROWS)])

    pltpu.sync_copy(src_hbm.at[w], sidx_all)
    plsc.subcore_barrier()

    tile_base = w * EPW

    def didx_load(i, dbuf, sem):
        ebase = pl.multiple_of(tile_base + i * CHUNK, 8)
        return pltpu.make_async_copy(dst_hbm.at[pl.ds(ebase, CHUNK)],
                                     dbuf, sem)

    def gather(i, buf, sem):
        return pltpu.make_async_copy(x_hbm.at[sidx_all.at[i]], buf, sem)

    def scatter(dbuf, buf):
        pltpu.sync_copy(buf, agg_sh.at[dbuf], add=True)

    didx_load(0, didx0, semi0).start()
    didx_load(1, didx1, semi1).start()
    gather(0, rows0, semg0).start()

    def pair(p, carry):
        i0 = 2 * p
        gather(i0 + 1, rows1, semg1).start()
        gather(i0, rows0, semg0).wait()
        didx_load(i0, didx0, semi0).wait()
        scatter(didx0, rows0)
        gather(i0 + 2, rows0, semg0).start()
        didx_load(i0 + 2, didx0, semi0).start()
        gather(i0 + 1, rows1, semg1).wait()
        didx_load(i0 + 1, didx1, semi1).wait()
        scatter(didx1, rows1)
        didx_load(jnp.minimum(i0 + 3, ITERS - 1), didx1, semi1).start()
        return carry

    lax.fori_loop(0, (ITERS - 1) // 2, pair, 0)
    gather(ITERS - 1, rows0, semg0).wait()
    didx_load(ITERS - 1, didx0, semi0).wait()
    scatter(didx0, rows0)
    didx_load(ITERS - 1, didx1, semi1).wait()

    plsc.subcore_barrier()

    @pl.when(s < NS - 1)
    def _exp_full():
        pltpu.sync_copy(agg_sh.at[pl.ds(rbase, ROWS_PER_TILE)],
                        parts_hbm.at[c, pl.ds(rbase, ROWS_PER_TILE)])

    @pl.when(s == NS - 1)
    def _exp_last():
        pltpu.sync_copy(agg_sh.at[pl.ds((NS - 1) * ROWS_PER_TILE,
                                        LAST_TILE_ROWS)],
                        parts_hbm.at[c, pl.ds((NS - 1) * ROWS_PER_TILE,
                                              LAST_TILE_ROWS)])


_sc_agg = pl.kernel(
    _sc_agg_body,
    out_type=jax.ShapeDtypeStruct((NC, NPAD, D), jnp.float32),
    mesh=plsc.VectorSubcoreMesh(core_axis_name="c", subcore_axis_name="s"),
    scratch_types=[
        pltpu.VMEM_SHARED((NPAD, D), jnp.float32),
        pltpu.VMEM((ITERS, CHUNK), jnp.int32),
        pltpu.VMEM((CHUNK,), jnp.int32),
        pltpu.VMEM((CHUNK,), jnp.int32),
        pltpu.VMEM((CHUNK, D), jnp.float32),
        pltpu.VMEM((CHUNK, D), jnp.float32),
        pltpu.SemaphoreType.DMA,
        pltpu.SemaphoreType.DMA,
        pltpu.SemaphoreType.DMA,
        pltpu.SemaphoreType.DMA,
    ],
)


BLK = 1000
NB = N // BLK


def _tc_mlp_bn_body(parts, x, W1, W2, b1, b2, gamma, beta, y, h_s, stat_s):
    p = pl.program_id(0)
    j = pl.program_id(1)

    @pl.when(p == 0)
    def _phase0():
        hin = parts[0] + parts[1] - x[...]
        m = jnp.dot(hin, W1[...], preferred_element_type=jnp.float32)
        m = jnp.maximum(m + b1[...], 0.0)
        h = jnp.dot(m, W2[...], preferred_element_type=jnp.float32)
        h = jnp.maximum(h + b2[...], 0.0)
        h_s[pl.ds(j * BLK, BLK), :] = h
        bs = jnp.sum(h, axis=0, keepdims=True)
        bq = jnp.sum(h * h, axis=0, keepdims=True)

        @pl.when(j == 0)
        def _init():
            stat_s[0:1, :] = bs
            stat_s[1:2, :] = bq

        @pl.when(j > 0)
        def _acc():
            stat_s[0:1, :] += bs
            stat_s[1:2, :] += bq

        y[...] = h

    @pl.when(p == 1)
    def _phase1():
        mean = stat_s[0:1, :] * (1.0 / N)
        var = stat_s[1:2, :] * (1.0 / N) - mean * mean
        rstd = lax.rsqrt(var + 1e-5)
        h = h_s[pl.ds(j * BLK, BLK), :]
        y[...] = (h - mean) * (rstd * gamma[...]) + beta[...]


def _tc_mlp_bn(parts, x, W1, W2, b1, b2, gamma, beta):
    row_spec = pl.BlockSpec((BLK, D), lambda p, j: (j, 0))
    mat_spec = pl.BlockSpec((D, D), lambda p, j: (0, 0))
    vec_spec = pl.BlockSpec((1, D), lambda p, j: (0, 0))
    parts_spec = pl.BlockSpec((NC, BLK, D), lambda p, j: (0, j, 0))
    return pl.pallas_call(
        _tc_mlp_bn_body,
        grid=(2, NB),
        in_specs=[parts_spec, row_spec, mat_spec, mat_spec,
                  vec_spec, vec_spec, vec_spec, vec_spec],
        out_specs=row_spec,
        out_shape=jax.ShapeDtypeStruct((N, D), jnp.float32),
        scratch_shapes=[
            pltpu.VMEM((N, D), jnp.float32),
            pltpu.VMEM((2, D), jnp.float32),
        ],
        compiler_params=pltpu.CompilerParams(
            dimension_semantics=("arbitrary", "arbitrary")),
    )(parts, x, W1, W2, b1, b2, gamma, beta)


def kernel(x, edge_index, batch, W1, b1, W2, b2, gamma, beta):
    src = edge_index[0].astype(jnp.int32).reshape(NW, ITERS, CHUNK)
    dst = edge_index[1].astype(jnp.int32)
    parts = _sc_agg(x, src, dst)
    return _tc_mlp_bn(parts, x, W1, W2,
                      b1.reshape(1, D), b2.reshape(1, D),
                      gamma.reshape(1, D), beta.reshape(1, D))
